# baseline (speedup 1.0000x reference)
import functools

import jax
import jax.numpy as jnp
from jax import lax
from jax.experimental import pallas as pl
from jax.experimental.pallas import tpu as pltpu
from jax.experimental.pallas import tpu_sc as plsc

B, L, D = 4096, 200, 64
NC, NS = 2, 16
NW = NC * NS
TOK = B * L
TPW = TOK // NW
T = 400
NCH = TPW // T
VPD = D // 16


def _body(emb_hbm, x_hbm, pos_hbm, out_hbm, idx_v, rows_v, pos_v, gsem):
    wid = lax.axis_index("s") * NC + lax.axis_index("c")
    base_w = wid * TPW
    pltpu.sync_copy(pos_hbm.at[pl.ds(0, L)], pos_v)

    @pl.loop(0, NCH)
    def _chunk(i):
        base = base_w + i * T
        pltpu.sync_copy(x_hbm.at[pl.ds(base, T)], idx_v)
        pltpu.async_copy(emb_hbm.at[idx_v], rows_v, gsem).wait()

        @pl.loop(0, L)
        def _add(j):
            for c in range(VPD):
                p = pos_v[j, pl.ds(c * 16, 16)]
                for r in range(T // L):
                    t = r * L + j
                    rows_v[t, pl.ds(c * 16, 16)] = (
                        rows_v[t, pl.ds(c * 16, 16)] + p
                    )

        pltpu.sync_copy(rows_v, out_hbm.at[pl.ds(base, T)])


@jax.jit
def kernel(x, embed_table, pos_table):
    x_flat = x.reshape(TOK).astype(jnp.int32)
    mesh = plsc.VectorSubcoreMesh(core_axis_name="c", subcore_axis_name="s",
                                  num_cores=NC, num_subcores=NS)
    out = pl.kernel(
        _body,
        out_type=jax.ShapeDtypeStruct((TOK, D), jnp.float32),
        mesh=mesh,
        compiler_params=pltpu.CompilerParams(use_tc_tiling_on_sc=False),
        scratch_types=[
            pltpu.VMEM((T,), jnp.int32),
            pltpu.VMEM((T, D), jnp.float32),
            pltpu.VMEM((L, D), jnp.float32),
            pltpu.SemaphoreType.DMA,
        ],
    )(embed_table, x_flat, pos_table)
    return out.reshape(B, L, D)
